# baseline (speedup 1.0000x reference)
import functools

import jax
import jax.numpy as jnp
from jax import lax
from jax.experimental import pallas as pl
from jax.experimental.pallas import tpu as pltpu
from jax.experimental.pallas import tpu_sc as plsc

N = 10000
E = 320000
D = 128
H = 16
C = 40

L = 16
NC = 2
NS = 16
NW = NC * NS
K = 128
NBUF = 8
T = NBUF * (-(-E // (NW * K * NBUF)))
E_PAD = T * NW * K
ACC_ROWS = 10112
RPS = ACC_ROWS // NS


def _spmm_sc(x, src3, dst3, zeros):
    mesh = plsc.VectorSubcoreMesh(
        core_axis_name="c", subcore_axis_name="s",
        num_cores=NC, num_subcores=NS)

    @functools.partial(
        pl.kernel,
        out_type=jax.ShapeDtypeStruct((NC, ACC_ROWS, L), jnp.float32),
        mesh=mesh,
        compiler_params=pltpu.CompilerParams(use_tc_tiling_on_sc=False),
        scratch_types=[
            pltpu.VMEM((T, K), jnp.int32),
            pltpu.VMEM((T, K), jnp.int32),
            pltpu.VMEM((NBUF, K, L), jnp.float32),
            pltpu.VMEM_SHARED((ACC_ROWS, L), jnp.float32),
            [pltpu.SemaphoreType.DMA] * NBUF,
            [pltpu.SemaphoreType.DMA] * NBUF,
        ],
    )
    def body(x_hbm, src_hbm, dst_hbm, z_hbm, out_hbm,
             src_v, dst_v, rows, acc, gsems, ssems):
        cid = lax.axis_index("c")
        sid = lax.axis_index("s")
        wid = sid * NC + cid
        base = sid * RPS
        pltpu.sync_copy(z_hbm.at[pl.ds(base, RPS)], acc.at[pl.ds(base, RPS)])
        pltpu.sync_copy(src_hbm.at[wid], src_v)
        pltpu.sync_copy(dst_hbm.at[wid], dst_v)
        plsc.subcore_barrier()

        for b in range(NBUF - 1):
            pltpu.async_copy(x_hbm.at[src_v.at[b]], rows.at[b], gsems[b])

        def outer(tg, carry):
            for b in range(NBUF):
                t = tg * NBUF + b
                nxt = t + NBUF - 1
                nb = (b - 1) % NBUF

                @pl.when((nxt < T) & (t > 0))
                def _():
                    pltpu.make_async_copy(rows.at[nb],
                                          acc.at[dst_v.at[t - 1]],
                                          ssems[nb]).wait()

                @pl.when(nxt < T)
                def _():
                    pltpu.async_copy(x_hbm.at[src_v.at[nxt]], rows.at[nb],
                                     gsems[nb])
                pltpu.make_async_copy(x_hbm.at[src_v.at[t]], rows.at[b],
                                      gsems[b]).wait()
                pltpu.async_copy(rows.at[b], acc.at[dst_v.at[t]], ssems[b],
                                 add=True)
            return carry

        lax.fori_loop(0, T // NBUF, outer, 0)
        for b in range(NBUF):
            pltpu.make_async_copy(rows.at[b],
                                  acc.at[dst_v.at[T - NBUF + b]],
                                  ssems[b]).wait()
        plsc.subcore_barrier()
        pltpu.sync_copy(acc.at[pl.ds(base, RPS)],
                        out_hbm.at[cid, pl.ds(base, RPS)])

    return body(x, src3, dst3, zeros)


def _mm1(features, W1):
    def body(f_ref, w_ref, o_ref):
        o_ref[...] = jnp.dot(f_ref[...], w_ref[...],
                             preferred_element_type=jnp.float32)
    return pl.pallas_call(
        body, out_shape=jax.ShapeDtypeStruct((N, H), jnp.float32),
    )(features, W1)


def _relu_sum(z):
    def body(z_ref, o_ref):
        o_ref[...] = jnp.maximum(z_ref[0] + z_ref[1], 0.0)
    return pl.pallas_call(
        body, out_shape=jax.ShapeDtypeStruct((ACC_ROWS, L), jnp.float32),
    )(z)


def _out_head(z, W2):
    def body(z_ref, w_ref, o_ref):
        q = jnp.dot(z_ref[0] + z_ref[1], w_ref[...],
                    preferred_element_type=jnp.float32)
        m = jnp.max(q, axis=1, keepdims=True)
        lse = m + jnp.log(jnp.sum(jnp.exp(q - m), axis=1, keepdims=True))
        o_ref[...] = (q - lse)[:N]
    return pl.pallas_call(
        body, out_shape=jax.ShapeDtypeStruct((N, C), jnp.float32),
    )(z, W2)


def kernel(features, edge_index, W1, W2):
    src = edge_index[0]
    dst = edge_index[1]
    pad = E_PAD - E
    src3 = jnp.concatenate([src, jnp.zeros((pad,), jnp.int32)]).reshape(NW, T, K)
    dst3 = jnp.concatenate([dst, jnp.full((pad,), N, jnp.int32)]).reshape(NW, T, K)
    zeros = jnp.zeros((ACC_ROWS, L), jnp.float32)

    y = _mm1(features, W1)
    z1 = _spmm_sc(y, src3, dst3, zeros)
    h = _relu_sum(z1)
    z2 = _spmm_sc(h, src3, dst3, zeros)
    return _out_head(z2, W2)


# speedup vs baseline: 1.0473x; 1.0473x over previous
import functools

import jax
import jax.numpy as jnp
from jax import lax
from jax.experimental import pallas as pl
from jax.experimental.pallas import tpu as pltpu
from jax.experimental.pallas import tpu_sc as plsc

N = 10000
E = 320000
D = 128
H = 16
C = 40

L = 16
NC = 2
NS = 16
NW = NC * NS
K = 128
NBUF = 8
T = NBUF * (-(-E // (NW * K * NBUF)))
E_PAD = T * NW * K
ACC_ROWS = 10112
RPS = ACC_ROWS // NS


def _spmm_sc(x, idx3):
    mesh = plsc.VectorSubcoreMesh(
        core_axis_name="c", subcore_axis_name="s",
        num_cores=NC, num_subcores=NS)

    @functools.partial(
        pl.kernel,
        out_type=jax.ShapeDtypeStruct((NC, ACC_ROWS, L), jnp.float32),
        mesh=mesh,
        compiler_params=pltpu.CompilerParams(use_tc_tiling_on_sc=False),
        scratch_types=[
            pltpu.VMEM((2, T, K), jnp.int32),
            pltpu.VMEM((NBUF, K, L), jnp.float32),
            pltpu.VMEM((RPS, L), jnp.float32),
            pltpu.VMEM_SHARED((ACC_ROWS, L), jnp.float32),
            [pltpu.SemaphoreType.DMA] * NBUF,
            [pltpu.SemaphoreType.DMA] * NBUF,
        ],
    )
    def body(x_hbm, idx_hbm, out_hbm,
             idx_v, rows, zbuf, acc, gsems, ssems):
        cid = lax.axis_index("c")
        sid = lax.axis_index("s")
        wid = sid * NC + cid
        base = sid * RPS
        pltpu.sync_copy(idx_hbm.at[wid], idx_v)
        src_v = idx_v.at[0]
        dst_v = idx_v.at[1]
        def zrow(i, carry):
            zbuf[i, :] = jnp.zeros((L,), jnp.float32)
            return carry
        lax.fori_loop(0, RPS, zrow, 0)
        pltpu.sync_copy(zbuf, acc.at[pl.ds(base, RPS)])
        plsc.subcore_barrier()

        for b in range(NBUF - 1):
            pltpu.async_copy(x_hbm.at[src_v.at[b]], rows.at[b], gsems[b])

        def outer(tg, carry):
            for b in range(NBUF):
                t = tg * NBUF + b
                nxt = t + NBUF - 1
                nb = (b - 1) % NBUF

                @pl.when((nxt < T) & (t > 0))
                def _():
                    pltpu.make_async_copy(rows.at[nb],
                                          acc.at[dst_v.at[t - 1]],
                                          ssems[nb]).wait()

                @pl.when(nxt < T)
                def _():
                    pltpu.async_copy(x_hbm.at[src_v.at[nxt]], rows.at[nb],
                                     gsems[nb])
                pltpu.make_async_copy(x_hbm.at[src_v.at[t]], rows.at[b],
                                      gsems[b]).wait()
                pltpu.async_copy(rows.at[b], acc.at[dst_v.at[t]], ssems[b],
                                 add=True)
            return carry

        lax.fori_loop(0, T // NBUF, outer, 0)
        for b in range(NBUF):
            pltpu.make_async_copy(rows.at[b],
                                  acc.at[dst_v.at[T - NBUF + b]],
                                  ssems[b]).wait()
        plsc.subcore_barrier()
        pltpu.sync_copy(acc.at[pl.ds(base, RPS)],
                        out_hbm.at[cid, pl.ds(base, RPS)])

    return body(x, idx3)


def _mm1(features, W1):
    def body(f_ref, w_ref, o_ref):
        o_ref[...] = jnp.dot(f_ref[...], w_ref[...],
                             preferred_element_type=jnp.float32)
    return pl.pallas_call(
        body, out_shape=jax.ShapeDtypeStruct((N, H), jnp.float32),
    )(features, W1)


def _relu_sum(z):
    def body(z_ref, o_ref):
        o_ref[...] = jnp.maximum(z_ref[0] + z_ref[1], 0.0)
    return pl.pallas_call(
        body, out_shape=jax.ShapeDtypeStruct((ACC_ROWS, L), jnp.float32),
    )(z)


def _out_head(z, W2):
    def body(z_ref, w_ref, o_ref):
        q = jnp.dot(z_ref[0] + z_ref[1], w_ref[...],
                    preferred_element_type=jnp.float32)
        m = jnp.max(q, axis=1, keepdims=True)
        lse = m + jnp.log(jnp.sum(jnp.exp(q - m), axis=1, keepdims=True))
        o_ref[...] = (q - lse)[:N]
    return pl.pallas_call(
        body, out_shape=jax.ShapeDtypeStruct((N, C), jnp.float32),
    )(z, W2)


def kernel(features, edge_index, W1, W2):
    src = edge_index[0]
    dst = edge_index[1]
    pad = E_PAD - E
    src3 = jnp.concatenate([src, jnp.zeros((pad,), jnp.int32)]).reshape(NW, T, K)
    dst3 = jnp.concatenate([dst, jnp.full((pad,), N, jnp.int32)]).reshape(NW, T, K)
    idx3 = jnp.stack([src3, dst3], axis=1)

    y = _mm1(features, W1)
    z1 = _spmm_sc(y, idx3)
    h = _relu_sum(z1)
    z2 = _spmm_sc(h, idx3)
    return _out_head(z2, W2)


# speedup vs baseline: 1.4600x; 1.3940x over previous
import functools

import jax
import jax.numpy as jnp
from jax import lax
from jax.experimental import pallas as pl
from jax.experimental.pallas import tpu as pltpu
from jax.experimental.pallas import tpu_sc as plsc

N = 10000
E = 320000
D = 128
H = 16
C = 40

L = 16
NC = 2
NS = 16
NW = NC * NS
K = 128
NBUF = 8
T = NBUF * (-(-E // (NW * K * NBUF)))
E_PAD = T * NW * K
ACC_ROWS = 10112
RPS = ACC_ROWS // NS


def _spmm_sc(x, idx3):
    mesh = plsc.VectorSubcoreMesh(
        core_axis_name="c", subcore_axis_name="s",
        num_cores=NC, num_subcores=NS)

    @functools.partial(
        pl.kernel,
        out_type=jax.ShapeDtypeStruct((NC, ACC_ROWS, L), jnp.float32),
        mesh=mesh,
        compiler_params=pltpu.CompilerParams(use_tc_tiling_on_sc=False),
        scratch_types=[
            pltpu.VMEM((2, T, K), jnp.int32),
            pltpu.VMEM((NBUF, K, L), jnp.float32),
            pltpu.VMEM((RPS, L), jnp.float32),
            pltpu.VMEM_SHARED((ACC_ROWS, L), jnp.float32),
            pltpu.VMEM_SHARED((N, L), jnp.float32),
            [pltpu.SemaphoreType.DMA] * NBUF,
            [pltpu.SemaphoreType.DMA] * NBUF,
        ],
    )
    def body(x_hbm, idx_hbm, out_hbm,
             idx_v, rows, zbuf, acc, x_sp, gsems, ssems):
        cid = lax.axis_index("c")
        sid = lax.axis_index("s")
        wid = sid * NC + cid
        base = sid * RPS
        xrows = N // NS
        pltpu.sync_copy(x_hbm.at[pl.ds(sid * xrows, xrows)],
                        x_sp.at[pl.ds(sid * xrows, xrows)])
        pltpu.sync_copy(idx_hbm.at[wid], idx_v)
        src_v = idx_v.at[0]
        dst_v = idx_v.at[1]
        def zrow(i, carry):
            zbuf[i, :] = jnp.zeros((L,), jnp.float32)
            return carry
        lax.fori_loop(0, RPS, zrow, 0)
        pltpu.sync_copy(zbuf, acc.at[pl.ds(base, RPS)])
        plsc.subcore_barrier()

        for b in range(NBUF - 1):
            pltpu.async_copy(x_sp.at[src_v.at[b]], rows.at[b], gsems[b])

        def outer(tg, carry):
            for b in range(NBUF):
                t = tg * NBUF + b
                nxt = t + NBUF - 1
                nb = (b - 1) % NBUF

                @pl.when((nxt < T) & (t > 0))
                def _():
                    pltpu.make_async_copy(rows.at[nb],
                                          acc.at[dst_v.at[t - 1]],
                                          ssems[nb]).wait()

                @pl.when(nxt < T)
                def _():
                    pltpu.async_copy(x_sp.at[src_v.at[nxt]], rows.at[nb],
                                     gsems[nb])
                pltpu.make_async_copy(x_sp.at[src_v.at[t]], rows.at[b],
                                      gsems[b]).wait()
                pltpu.async_copy(rows.at[b], acc.at[dst_v.at[t]], ssems[b],
                                 add=True)
            return carry

        lax.fori_loop(0, T // NBUF, outer, 0)
        for b in range(NBUF):
            pltpu.make_async_copy(rows.at[b],
                                  acc.at[dst_v.at[T - NBUF + b]],
                                  ssems[b]).wait()
        plsc.subcore_barrier()
        pltpu.sync_copy(acc.at[pl.ds(base, RPS)],
                        out_hbm.at[cid, pl.ds(base, RPS)])

    return body(x, idx3)


def _mm1(features, W1):
    def body(f_ref, w_ref, o_ref):
        o_ref[...] = jnp.dot(f_ref[...], w_ref[...],
                             preferred_element_type=jnp.float32)
    return pl.pallas_call(
        body, out_shape=jax.ShapeDtypeStruct((N, H), jnp.float32),
    )(features, W1)


def _relu_sum(z):
    def body(z_ref, o_ref):
        o_ref[...] = jnp.maximum(z_ref[0] + z_ref[1], 0.0)
    return pl.pallas_call(
        body, out_shape=jax.ShapeDtypeStruct((ACC_ROWS, L), jnp.float32),
    )(z)


def _out_head(z, W2):
    def body(z_ref, w_ref, o_ref):
        q = jnp.dot(z_ref[0] + z_ref[1], w_ref[...],
                    preferred_element_type=jnp.float32)
        m = jnp.max(q, axis=1, keepdims=True)
        lse = m + jnp.log(jnp.sum(jnp.exp(q - m), axis=1, keepdims=True))
        o_ref[...] = (q - lse)[:N]
    return pl.pallas_call(
        body, out_shape=jax.ShapeDtypeStruct((N, C), jnp.float32),
    )(z, W2)


def kernel(features, edge_index, W1, W2):
    src = edge_index[0]
    dst = edge_index[1]
    pad = E_PAD - E
    src3 = jnp.concatenate([src, jnp.zeros((pad,), jnp.int32)]).reshape(NW, T, K)
    dst3 = jnp.concatenate([dst, jnp.full((pad,), N, jnp.int32)]).reshape(NW, T, K)
    idx3 = jnp.stack([src3, dst3], axis=1)

    y = _mm1(features, W1)
    z1 = _spmm_sc(y, idx3)
    h = _relu_sum(z1)
    z2 = _spmm_sc(h, idx3)
    return _out_head(z2, W2)


# speedup vs baseline: 1.5979x; 1.0945x over previous
import functools

import jax
import jax.numpy as jnp
from jax import lax
from jax.experimental import pallas as pl
from jax.experimental.pallas import tpu as pltpu
from jax.experimental.pallas import tpu_sc as plsc

N = 10000
E = 320000
D = 128
H = 16
C = 40

L = 16
NC = 2
NS = 16
NW = NC * NS
K = 128
NBUF = 8
T = NBUF * (-(-E // (NW * K * NBUF)))
E_PAD = T * NW * K
ACC_ROWS = 10112
RPS = ACC_ROWS // NS


def _spmm_sc(x, idx3, fuse_relu_combine):
    mesh = plsc.VectorSubcoreMesh(
        core_axis_name="c", subcore_axis_name="s",
        num_cores=NC, num_subcores=NS)

    @functools.partial(
        pl.kernel,
        out_type=jax.ShapeDtypeStruct((NC, ACC_ROWS, L), jnp.float32),
        mesh=mesh,
        compiler_params=pltpu.CompilerParams(use_tc_tiling_on_sc=False),
        scratch_types=[
            pltpu.VMEM((2, T, K), jnp.int32),
            pltpu.VMEM((NBUF, K, L), jnp.float32),
            pltpu.VMEM((RPS, L), jnp.float32),
            pltpu.VMEM((RPS, L), jnp.float32),
            pltpu.VMEM((RPS, L), jnp.float32),
            pltpu.VMEM_SHARED((ACC_ROWS, L), jnp.float32),
            pltpu.VMEM_SHARED((N, L), jnp.float32),
            [pltpu.SemaphoreType.DMA] * NBUF,
            [pltpu.SemaphoreType.DMA] * NBUF,
        ],
    )
    def body(x_hbm, idx_hbm, out_hbm,
             idx_v, rows, zbuf, sta, stb, acc, x_sp, gsems, ssems):
        cid = lax.axis_index("c")
        sid = lax.axis_index("s")
        wid = sid * NC + cid
        base = sid * RPS
        xrows = N // NS
        if fuse_relu_combine:
            pltpu.sync_copy(x_hbm.at[0, pl.ds(sid * xrows, xrows)],
                            sta.at[pl.ds(0, xrows)])
            pltpu.sync_copy(x_hbm.at[1, pl.ds(sid * xrows, xrows)],
                            stb.at[pl.ds(0, xrows)])

            def rrow(i, carry):
                sta[i, :] = jnp.maximum(sta[i, :] + stb[i, :], 0.0)
                return carry
            lax.fori_loop(0, xrows, rrow, 0)
            pltpu.sync_copy(sta.at[pl.ds(0, xrows)],
                            x_sp.at[pl.ds(sid * xrows, xrows)])
        else:
            pltpu.sync_copy(x_hbm.at[pl.ds(sid * xrows, xrows)],
                            x_sp.at[pl.ds(sid * xrows, xrows)])
        pltpu.sync_copy(idx_hbm.at[wid], idx_v)
        src_v = idx_v.at[0]
        dst_v = idx_v.at[1]
        def zrow(i, carry):
            zbuf[i, :] = jnp.zeros((L,), jnp.float32)
            return carry
        lax.fori_loop(0, RPS, zrow, 0)
        pltpu.sync_copy(zbuf, acc.at[pl.ds(base, RPS)])
        plsc.subcore_barrier()

        for b in range(NBUF - 1):
            pltpu.async_copy(x_sp.at[src_v.at[b]], rows.at[b], gsems[b])

        def outer(tg, carry):
            for b in range(NBUF):
                t = tg * NBUF + b
                nxt = t + NBUF - 1
                nb = (b - 1) % NBUF

                @pl.when((nxt < T) & (t > 0))
                def _():
                    pltpu.make_async_copy(rows.at[nb],
                                          acc.at[dst_v.at[t - 1]],
                                          ssems[nb]).wait()

                @pl.when(nxt < T)
                def _():
                    pltpu.async_copy(x_sp.at[src_v.at[nxt]], rows.at[nb],
                                     gsems[nb])
                pltpu.make_async_copy(x_sp.at[src_v.at[t]], rows.at[b],
                                      gsems[b]).wait()
                pltpu.async_copy(rows.at[b], acc.at[dst_v.at[t]], ssems[b],
                                 add=True)
            return carry

        lax.fori_loop(0, T // NBUF, outer, 0)
        for b in range(NBUF):
            pltpu.make_async_copy(rows.at[b],
                                  acc.at[dst_v.at[T - NBUF + b]],
                                  ssems[b]).wait()
        plsc.subcore_barrier()
        pltpu.sync_copy(acc.at[pl.ds(base, RPS)],
                        out_hbm.at[cid, pl.ds(base, RPS)])

    return body(x, idx3)


def _mm1(features, W1):
    def body(f_ref, w_ref, o_ref):
        o_ref[...] = jnp.dot(f_ref[...], w_ref[...],
                             preferred_element_type=jnp.float32)
    return pl.pallas_call(
        body, out_shape=jax.ShapeDtypeStruct((N, H), jnp.float32),
    )(features, W1)


def _out_head(z, W2):
    def body(z_ref, w_ref, o_ref):
        q = jnp.dot(z_ref[0] + z_ref[1], w_ref[...],
                    preferred_element_type=jnp.float32)
        m = jnp.max(q, axis=1, keepdims=True)
        lse = m + jnp.log(jnp.sum(jnp.exp(q - m), axis=1, keepdims=True))
        o_ref[...] = (q - lse)[:N]
    return pl.pallas_call(
        body, out_shape=jax.ShapeDtypeStruct((N, C), jnp.float32),
    )(z, W2)


def kernel(features, edge_index, W1, W2):
    src = edge_index[0]
    dst = edge_index[1]
    pad = E_PAD - E
    src3 = jnp.concatenate([src, jnp.zeros((pad,), jnp.int32)]).reshape(NW, T, K)
    dst3 = jnp.concatenate([dst, jnp.full((pad,), N, jnp.int32)]).reshape(NW, T, K)
    idx3 = jnp.stack([src3, dst3], axis=1)

    y = _mm1(features, W1)
    z1 = _spmm_sc(y, idx3, False)
    z2 = _spmm_sc(z1, idx3, True)
    return _out_head(z2, W2)


# speedup vs baseline: 1.6834x; 1.0535x over previous
import functools

import jax
import jax.numpy as jnp
from jax import lax
from jax.experimental import pallas as pl
from jax.experimental.pallas import tpu as pltpu
from jax.experimental.pallas import tpu_sc as plsc

N = 10000
E = 320000
D = 128
H = 16
C = 40

L = 16
NC = 2
NS = 16
NW = NC * NS
K = 128
NBUF = 8
EPW = E // NW
T = NBUF * (-(-EPW // (K * NBUF)))
TAIL = T * K - EPW
ACC_ROWS = 10112
RPS = ACC_ROWS // NS


def _spmm_sc(x, edge_index, fuse_relu_combine):
    mesh = plsc.VectorSubcoreMesh(
        core_axis_name="c", subcore_axis_name="s",
        num_cores=NC, num_subcores=NS)

    @functools.partial(
        pl.kernel,
        out_type=jax.ShapeDtypeStruct((NC, ACC_ROWS, L), jnp.float32),
        mesh=mesh,
        compiler_params=pltpu.CompilerParams(use_tc_tiling_on_sc=False),
        scratch_types=[
            pltpu.VMEM((2, T * K), jnp.int32),
            pltpu.VMEM((NBUF, K, L), jnp.float32),
            pltpu.VMEM((RPS, L), jnp.float32),
            pltpu.VMEM((RPS, L), jnp.float32),
            pltpu.VMEM((RPS, L), jnp.float32),
            pltpu.VMEM_SHARED((ACC_ROWS, L), jnp.float32),
            pltpu.VMEM_SHARED((N, L), jnp.float32),
            [pltpu.SemaphoreType.DMA] * NBUF,
            [pltpu.SemaphoreType.DMA] * NBUF,
        ],
    )
    def body(x_hbm, idx_hbm, out_hbm,
             idx_v, rows, zbuf, sta, stb, acc, x_sp, gsems, ssems):
        cid = lax.axis_index("c")
        sid = lax.axis_index("s")
        wid = sid * NC + cid
        base = sid * RPS
        xrows = N // NS
        if fuse_relu_combine:
            pltpu.sync_copy(x_hbm.at[0, pl.ds(sid * xrows, xrows)],
                            sta.at[pl.ds(0, xrows)])
            pltpu.sync_copy(x_hbm.at[1, pl.ds(sid * xrows, xrows)],
                            stb.at[pl.ds(0, xrows)])

            def rrow(i, carry):
                sta[i, :] = jnp.maximum(sta[i, :] + stb[i, :], 0.0)
                return carry
            lax.fori_loop(0, xrows, rrow, 0)
            pltpu.sync_copy(sta.at[pl.ds(0, xrows)],
                            x_sp.at[pl.ds(sid * xrows, xrows)])
        else:
            pltpu.sync_copy(x_hbm.at[pl.ds(sid * xrows, xrows)],
                            x_sp.at[pl.ds(sid * xrows, xrows)])
        pltpu.sync_copy(idx_hbm.at[0, pl.ds(wid * EPW, EPW)],
                        idx_v.at[0, pl.ds(0, EPW)])
        pltpu.sync_copy(idx_hbm.at[1, pl.ds(wid * EPW, EPW)],
                        idx_v.at[1, pl.ds(0, EPW)])
        for j in range(EPW, T * K, L):
            idx_v[0, pl.ds(j, L)] = jnp.zeros((L,), jnp.int32)
            idx_v[1, pl.ds(j, L)] = jnp.full((L,), N, jnp.int32)

        def src_v(t):
            return idx_v.at[0, pl.ds(t * K, K)]

        def dst_v(t):
            return idx_v.at[1, pl.ds(t * K, K)]
        def zrow(i, carry):
            zbuf[i, :] = jnp.zeros((L,), jnp.float32)
            return carry
        lax.fori_loop(0, RPS, zrow, 0)
        pltpu.sync_copy(zbuf, acc.at[pl.ds(base, RPS)])
        plsc.subcore_barrier()

        for b in range(NBUF - 1):
            pltpu.async_copy(x_sp.at[src_v(b)], rows.at[b], gsems[b])

        def outer(tg, carry):
            for b in range(NBUF):
                t = tg * NBUF + b
                nxt = t + NBUF - 1
                nb = (b - 1) % NBUF

                @pl.when((nxt < T) & (t > 0))
                def _():
                    pltpu.make_async_copy(rows.at[nb],
                                          acc.at[dst_v(t - 1)],
                                          ssems[nb]).wait()

                @pl.when(nxt < T)
                def _():
                    pltpu.async_copy(x_sp.at[src_v(nxt)], rows.at[nb],
                                     gsems[nb])
                pltpu.make_async_copy(x_sp.at[src_v(t)], rows.at[b],
                                      gsems[b]).wait()
                pltpu.async_copy(rows.at[b], acc.at[dst_v(t)], ssems[b],
                                 add=True)
            return carry

        lax.fori_loop(0, T // NBUF, outer, 0)
        for b in range(NBUF):
            pltpu.make_async_copy(rows.at[b],
                                  acc.at[dst_v(T - NBUF + b)],
                                  ssems[b]).wait()
        plsc.subcore_barrier()
        pltpu.sync_copy(acc.at[pl.ds(base, RPS)],
                        out_hbm.at[cid, pl.ds(base, RPS)])

    return body(x, edge_index)


def _mm1(features, W1):
    def body(f_ref, w_ref, o_ref):
        o_ref[...] = jnp.dot(f_ref[...], w_ref[...],
                             preferred_element_type=jnp.float32)
    return pl.pallas_call(
        body, out_shape=jax.ShapeDtypeStruct((N, H), jnp.float32),
    )(features, W1)


def _out_head(z, W2):
    def body(z_ref, w_ref, o_ref):
        q = jnp.dot(z_ref[0] + z_ref[1], w_ref[...],
                    preferred_element_type=jnp.float32)
        m = jnp.max(q, axis=1, keepdims=True)
        lse = m + jnp.log(jnp.sum(jnp.exp(q - m), axis=1, keepdims=True))
        o_ref[...] = (q - lse)[:N]
    return pl.pallas_call(
        body, out_shape=jax.ShapeDtypeStruct((N, C), jnp.float32),
    )(z, W2)


def kernel(features, edge_index, W1, W2):
    y = _mm1(features, W1)
    z1 = _spmm_sc(y, edge_index, False)
    z2 = _spmm_sc(z1, edge_index, True)
    return _out_head(z2, W2)


# speedup vs baseline: 1.8019x; 1.0704x over previous
import functools

import jax
import jax.numpy as jnp
from jax import lax
from jax.experimental import pallas as pl
from jax.experimental.pallas import tpu as pltpu
from jax.experimental.pallas import tpu_sc as plsc

N = 10000
E = 320000
D = 128
H = 16
C = 40

L = 16
NC = 2
NS = 16
NW = NC * NS
K = 128
NBUF = 8
EPW = E // NW
T = NBUF * (-(-EPW // (K * NBUF)))
TAIL = T * K - EPW
ACC_ROWS = 10112
RPS = ACC_ROWS // NS


def _spmm_sc(x, edge_index, fuse_relu_combine):
    mesh = plsc.VectorSubcoreMesh(
        core_axis_name="c", subcore_axis_name="s",
        num_cores=NC, num_subcores=NS)

    @functools.partial(
        pl.kernel,
        out_type=jax.ShapeDtypeStruct((NC, ACC_ROWS, L), jnp.float32),
        mesh=mesh,
        compiler_params=pltpu.CompilerParams(use_tc_tiling_on_sc=False),
        scratch_types=[
            pltpu.VMEM((2, T * K), jnp.int32),
            pltpu.VMEM((NBUF, K, L), jnp.float32),
            pltpu.VMEM((RPS, L), jnp.float32),
            pltpu.VMEM((RPS, L), jnp.float32),
            pltpu.VMEM((RPS, L), jnp.float32),
            pltpu.VMEM_SHARED((ACC_ROWS, L), jnp.float32),
            pltpu.VMEM_SHARED((N, L), jnp.float32),
            [pltpu.SemaphoreType.DMA] * NBUF,
            [pltpu.SemaphoreType.DMA] * NBUF,
        ],
    )
    def body(x_hbm, idx_hbm, out_hbm,
             idx_v, rows, zbuf, sta, stb, acc, x_sp, gsems, ssems):
        cid = lax.axis_index("c")
        sid = lax.axis_index("s")
        wid = sid * NC + cid
        base = sid * RPS
        xrows = N // NS
        src_ss = idx_hbm.at[pl.ds(wid * EPW, EPW)]
        dst_ss = idx_hbm.at[pl.ds(E + wid * EPW, EPW)]
        pltpu.async_copy(src_ss, idx_v.at[0, pl.ds(0, EPW)], gsems[0])
        pltpu.async_copy(dst_ss, idx_v.at[1, pl.ds(0, EPW)], gsems[1])
        if fuse_relu_combine:
            xa = x_hbm.at[0, pl.ds(sid * xrows, xrows)]
            xb = x_hbm.at[1, pl.ds(sid * xrows, xrows)]
            pltpu.async_copy(xa, sta.at[pl.ds(0, xrows)], gsems[2])
            pltpu.async_copy(xb, stb.at[pl.ds(0, xrows)], gsems[3])
        else:
            xa = x_hbm.at[pl.ds(sid * xrows, xrows)]
            pltpu.async_copy(xa, x_sp.at[pl.ds(sid * xrows, xrows)], gsems[2])
        for j in range(EPW, T * K, L):
            idx_v[0, pl.ds(j, L)] = jnp.zeros((L,), jnp.int32)
            idx_v[1, pl.ds(j, L)] = jnp.full((L,), N, jnp.int32)

        def src_v(t):
            return idx_v.at[0, pl.ds(t * K, K)]

        def dst_v(t):
            return idx_v.at[1, pl.ds(t * K, K)]
        def zrow(i, carry):
            zbuf[i, :] = jnp.zeros((L,), jnp.float32)
            return carry
        lax.fori_loop(0, RPS, zrow, 0)
        pltpu.make_async_copy(src_ss, idx_v.at[0, pl.ds(0, EPW)],
                              gsems[0]).wait()
        pltpu.make_async_copy(dst_ss, idx_v.at[1, pl.ds(0, EPW)],
                              gsems[1]).wait()
        if fuse_relu_combine:
            pltpu.make_async_copy(xa, sta.at[pl.ds(0, xrows)],
                                  gsems[2]).wait()
            pltpu.make_async_copy(xb, stb.at[pl.ds(0, xrows)],
                                  gsems[3]).wait()

            def rrow(i, carry):
                sta[i, :] = jnp.maximum(sta[i, :] + stb[i, :], 0.0)
                return carry
            lax.fori_loop(0, xrows, rrow, 0)
            pltpu.sync_copy(sta.at[pl.ds(0, xrows)],
                            x_sp.at[pl.ds(sid * xrows, xrows)])
        else:
            pltpu.make_async_copy(xa, x_sp.at[pl.ds(sid * xrows, xrows)],
                                  gsems[2]).wait()
        pltpu.sync_copy(zbuf, acc.at[pl.ds(base, RPS)])
        plsc.subcore_barrier()

        for b in range(NBUF - 1):
            pltpu.async_copy(x_sp.at[src_v(b)], rows.at[b], gsems[b])

        def outer(tg, carry):
            for b in range(NBUF):
                t = tg * NBUF + b
                nxt = t + NBUF - 1
                nb = (b - 1) % NBUF

                @pl.when((nxt < T) & (t > 0))
                def _():
                    pltpu.make_async_copy(rows.at[nb],
                                          acc.at[dst_v(t - 1)],
                                          ssems[nb]).wait()

                @pl.when(nxt < T)
                def _():
                    pltpu.async_copy(x_sp.at[src_v(nxt)], rows.at[nb],
                                     gsems[nb])
                pltpu.make_async_copy(x_sp.at[src_v(t)], rows.at[b],
                                      gsems[b]).wait()
                pltpu.async_copy(rows.at[b], acc.at[dst_v(t)], ssems[b],
                                 add=True)
            return carry

        lax.fori_loop(0, T // NBUF, outer, 0)
        for b in range(NBUF):
            pltpu.make_async_copy(rows.at[b],
                                  acc.at[dst_v(T - NBUF + b)],
                                  ssems[b]).wait()
        plsc.subcore_barrier()
        pltpu.sync_copy(acc.at[pl.ds(base, RPS)],
                        out_hbm.at[cid, pl.ds(base, RPS)])

    return body(x, edge_index)


def _mm1(features, W1):
    def body(f_ref, w_ref, o_ref):
        o_ref[...] = jnp.dot(f_ref[...], w_ref[...],
                             preferred_element_type=jnp.float32)
    return pl.pallas_call(
        body, out_shape=jax.ShapeDtypeStruct((N, H), jnp.float32),
    )(features, W1)


def _out_head(z, W2):
    def body(z_ref, w_ref, o_ref):
        q = jnp.dot(z_ref[0] + z_ref[1], w_ref[...],
                    preferred_element_type=jnp.float32)
        m = jnp.max(q, axis=1, keepdims=True)
        lse = m + jnp.log(jnp.sum(jnp.exp(q - m), axis=1, keepdims=True))
        o_ref[...] = (q - lse)[:N]
    return pl.pallas_call(
        body, out_shape=jax.ShapeDtypeStruct((N, C), jnp.float32),
    )(z, W2)


def kernel(features, edge_index, W1, W2):
    edge_flat = edge_index.reshape(2 * E)
    y = _mm1(features, W1)
    z1 = _spmm_sc(y, edge_flat, False)
    z2 = _spmm_sc(z1, edge_flat, True)
    return _out_head(z2, W2)
